# Initial kernel scaffold; baseline (speedup 1.0000x reference)
import jax
import jax.numpy as jnp
from jax.experimental import pallas as pl


def kernel(x, edge_index, W1, b1, W2, b2, Wfc, bfc):
    raise NotImplementedError("write your pallas kernel here")


# speedup vs baseline: 16.9496x; 16.9496x over previous
import jax
import jax.numpy as jnp
from jax import lax
from jax.experimental import pallas as pl
from jax.experimental.pallas import tpu as pltpu
from jax.experimental.pallas import tpu_sc as plsc

N = 50000
E = 800000
NS = 16
NW = 32
CH = 128
EPW = E // NW
NCHUNK = 196
EPW_PAD = NCHUNK * CH
NPAD = 50176
RPT = NPAD // NS
BLK = 2000
F32 = jnp.float32
_P = lax.Precision.HIGHEST

_mesh = plsc.VectorSubcoreMesh(core_axis_name="c", subcore_axis_name="s")
_sc_params = pltpu.CompilerParams(use_tc_tiling_on_sc=False)


def _sds(*shape):
    return jax.ShapeDtypeStruct(shape, F32)


def _deg_body(dstp, zeros4, ones4, dega, degb, stage, ones_v, acc):
    c = lax.axis_index("c")
    s = lax.axis_index("s")
    slc = pl.ds(s * RPT, RPT)
    pltpu.sync_copy(zeros4.at[slc], acc.at[slc])
    pltpu.sync_copy(ones4, ones_v)
    plsc.subcore_barrier()
    w = c * NS + s
    pltpu.sync_copy(dstp.at[w], stage)

    @pl.loop(0, NCHUNK)
    def _(j):
        pltpu.sync_copy(ones_v, acc.at[stage.at[j]], add=True)

    plsc.subcore_barrier()

    @pl.when(c == 0)
    def _():
        pltpu.sync_copy(acc.at[slc], dega.at[slc])

    @pl.when(c == 1)
    def _():
        pltpu.sync_copy(acc.at[slc], degb.at[slc])


_deg_kernel = pl.kernel(
    _deg_body,
    out_type=[_sds(NPAD, 4), _sds(NPAD, 4)],
    mesh=_mesh,
    compiler_params=_sc_params,
    scratch_types=[
        pltpu.VMEM((NCHUNK, CH), jnp.int32),
        pltpu.VMEM((CH, 4), F32),
        pltpu.VMEM_SHARED((NPAD, 4), F32),
    ],
)


def _agg1_body(srcp, dstp, xn, zeros4, agg1a, agg1b, sstage, dstage, rows, acc):
    c = lax.axis_index("c")
    s = lax.axis_index("s")
    slc = pl.ds(s * RPT, RPT)
    pltpu.sync_copy(zeros4.at[slc], acc.at[slc])
    plsc.subcore_barrier()
    w = c * NS + s
    pltpu.sync_copy(srcp.at[w], sstage)
    pltpu.sync_copy(dstp.at[w], dstage)

    @pl.loop(0, NCHUNK)
    def _(j):
        pltpu.sync_copy(xn.at[sstage.at[j]], rows)
        pltpu.sync_copy(rows, acc.at[dstage.at[j]], add=True)

    plsc.subcore_barrier()

    @pl.when(c == 0)
    def _():
        pltpu.sync_copy(acc.at[slc], agg1a.at[slc])

    @pl.when(c == 1)
    def _():
        pltpu.sync_copy(acc.at[slc], agg1b.at[slc])


_agg1_kernel = pl.kernel(
    _agg1_body,
    out_type=[_sds(NPAD, 4), _sds(NPAD, 4)],
    mesh=_mesh,
    compiler_params=_sc_params,
    scratch_types=[
        pltpu.VMEM((NCHUNK, CH), jnp.int32),
        pltpu.VMEM((NCHUNK, CH), jnp.int32),
        pltpu.VMEM((CH, 4), F32),
        pltpu.VMEM_SHARED((NPAD, 4), F32),
    ],
)


def _agg2_body(srcp, dstp, tq0, tq1, tq2, tq3, zeros16, a2q0, a2q1, a2q2, a2q3,
               sstage, dstage, rows, acc):
    c = lax.axis_index("c")
    s = lax.axis_index("s")
    slc = pl.ds(s * RPT, RPT)

    def run(table, out):
        pltpu.sync_copy(zeros16.at[slc], acc.at[slc])
        plsc.subcore_barrier()

        @pl.loop(0, 2)
        def _(b):
            w = 2 * s + b
            pltpu.sync_copy(srcp.at[w], sstage)
            pltpu.sync_copy(dstp.at[w], dstage)

            @pl.loop(0, NCHUNK)
            def _(j):
                pltpu.sync_copy(table.at[sstage.at[j]], rows)
                pltpu.sync_copy(rows, acc.at[dstage.at[j]], add=True)

        plsc.subcore_barrier()
        pltpu.sync_copy(acc.at[slc], out.at[slc])
        plsc.subcore_barrier()

    @pl.when(c == 0)
    def _():
        run(tq0, a2q0)
        run(tq1, a2q1)

    @pl.when(c == 1)
    def _():
        run(tq2, a2q2)
        run(tq3, a2q3)


_agg2_kernel = pl.kernel(
    _agg2_body,
    out_type=[_sds(NPAD, 16)] * 4,
    mesh=_mesh,
    compiler_params=_sc_params,
    scratch_types=[
        pltpu.VMEM((NCHUNK, CH), jnp.int32),
        pltpu.VMEM((NCHUNK, CH), jnp.int32),
        pltpu.VMEM((CH, 16), F32),
        pltpu.VMEM_SHARED((NPAD, 16), F32),
    ],
)


def _tc1_body(dega, degb, x, dis, xn):
    d = dega[:, 0:1] + degb[:, 0:1] + 1.0
    dv = lax.rsqrt(d)
    dis[...] = dv
    xb = x[...] * dv
    xn[...] = jnp.concatenate([xb, jnp.zeros((BLK, 1), F32)], axis=1)


def _tc1(dega, degb, x):
    return pl.pallas_call(
        _tc1_body,
        grid=(N // BLK,),
        in_specs=[
            pl.BlockSpec((BLK, 4), lambda b: (b, 0)),
            pl.BlockSpec((BLK, 4), lambda b: (b, 0)),
            pl.BlockSpec((BLK, 3), lambda b: (b, 0)),
        ],
        out_specs=[
            pl.BlockSpec((BLK, 1), lambda b: (b, 0)),
            pl.BlockSpec((BLK, 4), lambda b: (b, 0)),
        ],
        out_shape=[_sds(N, 1), _sds(N, 4)],
    )(dega, degb, x)


def _tc2_body(a1a, a1b, xn, dis, w1p, b1, w2, tq0, tq1, tq2, tq3):
    a = (a1a[...] + a1b[...] + xn[...]) * dis[...]
    h1 = jnp.maximum(jnp.dot(a, w1p[...], precision=_P) + b1[...], 0.0)
    t = jnp.dot(h1, w2[...], precision=_P) * dis[...]
    tq0[...] = t[:, 0:16]
    tq1[...] = t[:, 16:32]
    tq2[...] = t[:, 32:48]
    tq3[...] = t[:, 48:64]


def _tc2(a1a, a1b, xn, dis, w1p, b1, w2):
    return pl.pallas_call(
        _tc2_body,
        grid=(N // BLK,),
        in_specs=[
            pl.BlockSpec((BLK, 4), lambda b: (b, 0)),
            pl.BlockSpec((BLK, 4), lambda b: (b, 0)),
            pl.BlockSpec((BLK, 4), lambda b: (b, 0)),
            pl.BlockSpec((BLK, 1), lambda b: (b, 0)),
            pl.BlockSpec((4, 128), lambda b: (0, 0)),
            pl.BlockSpec((1, 128), lambda b: (0, 0)),
            pl.BlockSpec((128, 64), lambda b: (0, 0)),
        ],
        out_specs=[pl.BlockSpec((BLK, 16), lambda b: (b, 0))] * 4,
        out_shape=[_sds(N, 16)] * 4,
    )(a1a, a1b, xn, dis, w1p, b1, w2)


def _tc3_body(a0, a1, a2, a3, t0, t1, t2, t3, dis, b2, wfc, bfc, out, acc):
    b = pl.program_id(0)

    @pl.when(b == 0)
    def _():
        acc[...] = jnp.zeros((1, 64), F32)

    h = jnp.concatenate(
        [a0[...] + t0[...], a1[...] + t1[...],
         a2[...] + t2[...], a3[...] + t3[...]], axis=1)
    h = jnp.maximum(h * dis[...] + b2[...], 0.0)
    acc[...] += jnp.sum(h, axis=0, keepdims=True)

    @pl.when(b == N // BLK - 1)
    def _():
        g = acc[...] * (1.0 / N)
        out[...] = jnp.dot(g, wfc[...], precision=_P) + bfc[...]


def _tc3(aq, tq, dis, b2, wfc, bfc):
    return pl.pallas_call(
        _tc3_body,
        grid=(N // BLK,),
        in_specs=(
            [pl.BlockSpec((BLK, 16), lambda b: (b, 0))] * 8
            + [
                pl.BlockSpec((BLK, 1), lambda b: (b, 0)),
                pl.BlockSpec((1, 64), lambda b: (0, 0)),
                pl.BlockSpec((64, 3), lambda b: (0, 0)),
                pl.BlockSpec((1, 3), lambda b: (0, 0)),
            ]
        ),
        out_specs=pl.BlockSpec((1, 3), lambda b: (0, 0)),
        out_shape=_sds(1, 3),
        scratch_shapes=[pltpu.VMEM((1, 64), F32)],
    )(*aq, *tq, dis, b2, wfc, bfc)


def kernel(x, edge_index, W1, b1, W2, b2, Wfc, bfc):
    wids = jnp.arange(NW, dtype=jnp.int32)
    src_fill = jnp.broadcast_to(((wids * 997) % N)[:, None], (NW, EPW_PAD - EPW))
    dst_fill = jnp.broadcast_to((N + (wids % 8))[:, None], (NW, EPW_PAD - EPW))
    srcp = jnp.concatenate([edge_index[0].reshape(NW, EPW), src_fill], axis=1)
    dstp = jnp.concatenate([edge_index[1].reshape(NW, EPW), dst_fill], axis=1)
    srcp = srcp.reshape(NW, NCHUNK, CH)
    dstp = dstp.reshape(NW, NCHUNK, CH)

    zeros4 = jnp.zeros((NPAD, 4), F32)
    zeros16 = jnp.zeros((NPAD, 16), F32)
    ones4 = jnp.ones((CH, 4), F32)
    w1p = jnp.concatenate([W1, jnp.zeros((1, 128), F32)], axis=0)

    dega, degb = _deg_kernel(dstp, zeros4, ones4)
    dis, xn = _tc1(dega, degb, x)
    a1a, a1b = _agg1_kernel(srcp, dstp, xn, zeros4)
    tq = _tc2(a1a, a1b, xn, dis, w1p, b1.reshape(1, 128), W2)
    aq = _agg2_kernel(srcp, dstp, *tq, zeros16)
    out = _tc3(aq, tq, dis, b2.reshape(1, 64), Wfc, bfc.reshape(1, 3))
    return out.reshape(3)
